# speedup vs baseline: 2.9717x; 2.9717x over previous
import jax
import jax.numpy as jnp
from jax import lax
from jax.experimental import pallas as pl
from jax.experimental.pallas import tpu as pltpu
from jax.experimental.pallas import tpu_sc as plsc

VOCAB = 100000
D = 128
B = 4096
N = 50

NC = 2
NS = 16
NW = NC * NS

B_TOT = B * N
B_PER_W = B_TOT // NW
CHUNK = 128
N_CHUNKS = B_PER_W // CHUNK


def _body(table_hbm, idx_hbm, out_hbm, idx_v, rows_v, sem):
    wid = lax.axis_index("s") * NC + lax.axis_index("c")
    pltpu.sync_copy(idx_hbm.at[wid], idx_v)

    def step(j, carry):
        pltpu.async_copy(table_hbm.at[idx_v.at[j]], rows_v, sem).wait()
        pltpu.sync_copy(rows_v, out_hbm.at[wid, j])
        return carry

    lax.fori_loop(0, N_CHUNKS, step, 0)


@jax.jit
def kernel(x, table):
    idx = x.reshape(NW, N_CHUNKS, CHUNK).astype(jnp.int32)
    mesh = plsc.VectorSubcoreMesh(core_axis_name="c", subcore_axis_name="s")
    out = pl.kernel(
        _body,
        out_type=jax.ShapeDtypeStruct((NW, N_CHUNKS, CHUNK, D), jnp.float32),
        mesh=mesh,
        scratch_types=[
            pltpu.VMEM((N_CHUNKS, CHUNK), jnp.int32),
            pltpu.VMEM((CHUNK, D), jnp.float32),
            pltpu.SemaphoreType.DMA,
        ],
    )(table, idx)
    return out.reshape(B, N, D)


# speedup vs baseline: 3.3027x; 1.1114x over previous
import jax
import jax.numpy as jnp
from jax import lax
from jax.experimental import pallas as pl
from jax.experimental.pallas import tpu as pltpu
from jax.experimental.pallas import tpu_sc as plsc

VOCAB = 100000
D = 128
B = 4096
N = 50

NC = 2
NS = 16
NW = NC * NS

B_TOT = B * N
B_PER_W = B_TOT // NW
CHUNK = 128
N_CHUNKS = B_PER_W // CHUNK
NBUF = 5
ROUNDS = N_CHUNKS // NBUF


def _body(table_hbm, idx_hbm, out_hbm, idx_v, rows_v, gsem, wsem):
    wid = lax.axis_index("s") * NC + lax.axis_index("c")
    pltpu.sync_copy(idx_hbm.at[wid], idx_v)

    def gather(j, b):
        pltpu.async_copy(table_hbm.at[idx_v.at[j]], rows_v.at[b], gsem.at[b])

    def wait_gather(b):
        pltpu.make_async_copy(
            table_hbm.at[idx_v.at[0]], rows_v.at[b], gsem.at[b]).wait()

    def write(j, b):
        pltpu.async_copy(rows_v.at[b], out_hbm.at[wid, j], wsem.at[b])

    def wait_write(b):
        pltpu.make_async_copy(
            rows_v.at[b], out_hbm.at[wid, 0], wsem.at[b]).wait()

    for b in range(NBUF):
        gather(b, b)

    def round_body(r, carry):
        j0 = r * NBUF
        for b in range(NBUF):
            wait_gather(b)
            write(j0 + b, b)
        for b in range(NBUF):
            wait_write(b)
            gather(j0 + NBUF + b, b)
        return carry

    lax.fori_loop(0, ROUNDS - 1, round_body, 0)

    j0 = (ROUNDS - 1) * NBUF
    for b in range(NBUF):
        wait_gather(b)
        write(j0 + b, b)
    for b in range(NBUF):
        wait_write(b)


@jax.jit
def kernel(x, table):
    idx = x.reshape(NW, N_CHUNKS, CHUNK).astype(jnp.int32)
    mesh = plsc.VectorSubcoreMesh(core_axis_name="c", subcore_axis_name="s")
    out = pl.kernel(
        _body,
        out_type=jax.ShapeDtypeStruct((NW, N_CHUNKS, CHUNK, D), jnp.float32),
        mesh=mesh,
        scratch_types=[
            pltpu.VMEM((N_CHUNKS, CHUNK), jnp.int32),
            pltpu.VMEM((NBUF, CHUNK, D), jnp.float32),
            pltpu.SemaphoreType.DMA((NBUF,)),
            pltpu.SemaphoreType.DMA((NBUF,)),
        ],
    )(table, idx)
    return out.reshape(B, N, D)
